# baseline (speedup 1.0000x reference)
import math

import jax
import jax.numpy as jnp
from jax.experimental import pallas as pl
from jax.experimental.pallas import tpu as pltpu

_BC = 131072
_NSLOT = 3


def kernel(mem, vals, idx):
    limit, feat = mem.shape
    batch = vals.shape[0]
    mem_t = mem.T
    vals_t = vals.T
    bc = min(_BC, math.gcd(limit, batch))
    nb = limit // bc
    nvb = batch // bc

    idx32 = jnp.asarray(idx, dtype=jnp.int32)
    idx_arr = jnp.atleast_1d(idx32)

    def main_body(sp_ref, mem_ref, vals_ref, out_ref, vbuf, sems):
        i = pl.program_id(0)
        start = pl.multiple_of(sp_ref[0], 128)

        def interior(j):
            return (j * bc >= start) & ((j + 1) * bc <= start + batch)

        def vdma(j, slot):
            src0 = pl.multiple_of(j * bc - start, 128)
            return pltpu.make_async_copy(
                vals_ref.at[:, pl.ds(src0, bc)], vbuf.at[slot], sems.at[slot]
            )

        @pl.when(interior(i) & (i == 0))
        def _():
            vdma(i, i % _NSLOT).start()

        nxt = i + 1

        @pl.when((nxt < nb) & interior(nxt))
        def _():
            vdma(nxt, nxt % _NSLOT).start()

        @pl.when(interior(i))
        def _():
            vdma(i, i % _NSLOT).wait()
            out_ref[...] = vbuf[i % _NSLOT]

        @pl.when(jnp.logical_not(interior(i)))
        def _():
            out_ref[...] = mem_ref[...]

    def frozen_map(i, sp_ref):
        start = sp_ref[0]
        ws = start // bc
        inside = (i * bc >= start) & ((i + 1) * bc <= start + batch)
        return (0, jnp.where(inside, ws, i))

    filled = pl.pallas_call(
        main_body,
        grid_spec=pltpu.PrefetchScalarGridSpec(
            num_scalar_prefetch=1,
            grid=(nb,),
            in_specs=[
                pl.BlockSpec((feat, bc), frozen_map),
                pl.BlockSpec(memory_space=pltpu.MemorySpace.HBM),
            ],
            out_specs=pl.BlockSpec((feat, bc), lambda i, sp_ref: (0, i)),
            scratch_shapes=[
                pltpu.VMEM((_NSLOT, feat, bc), mem.dtype),
                pltpu.SemaphoreType.DMA((_NSLOT,)),
            ],
        ),
        out_shape=jax.ShapeDtypeStruct((feat, limit), mem.dtype),
    )(idx_arr, mem_t, vals_t)

    def fix_body(idx_ref, src_ref, vblk_ref, out_ref, sem):
        i = pl.program_id(0)
        start = pl.multiple_of(idx_ref[0], 128)
        dst = start + i * (batch - bc)
        cp = pltpu.make_async_copy(
            vblk_ref, out_ref.at[:, pl.ds(dst, bc)], sem
        )
        cp.start()
        cp.wait()

    new_mem_t = pl.pallas_call(
        fix_body,
        grid=(2,),
        in_specs=[
            pl.BlockSpec(memory_space=pltpu.MemorySpace.SMEM),
            pl.BlockSpec(memory_space=pltpu.MemorySpace.HBM),
            pl.BlockSpec((feat, bc), lambda i: (0, i * (nvb - 1))),
        ],
        out_specs=pl.BlockSpec(memory_space=pltpu.MemorySpace.HBM),
        out_shape=jax.ShapeDtypeStruct((feat, limit), mem.dtype),
        input_output_aliases={1: 0},
        scratch_shapes=[pltpu.SemaphoreType.DMA],
    )(idx_arr, filled, vals_t)

    new_idx = idx32 + batch
    return (new_mem_t.T, new_idx)
